# baseline (device time: 73880 ns/iter reference)
import os

import jax
import jax.numpy as jnp
from jax import lax
from jax.experimental import pallas as pl
from jax.experimental.pallas import tpu as pltpu

N_DEV = 4
SQ = 1024
SKV = 1024
HQ_SHARD = 8
DH = 128
BLK = 64
SCALE = 0.08838834764831843
N_CHUNKS = 2 * N_DEV
CHUNK = SQ // N_CHUNKS
N_HOPS = 2 * (N_DEV - 1)
P2 = 2 * CHUNK

_SKIP_COMM = bool(os.environ.get("SKIP_COMM"))


def kernel(x, Wq, K_ext, V_ext, Wo):
    x2 = x[0]

    def body(x_ref, wq_ref, kext_ref, vext_ref, wo_ref, out_ref,
             acc_ref, comm_ref, send_sems, recv_sems, ctx_ref, xp_ref,
             wqb_ref, wob_ref, kf_ref, vf_ref, kb_ref, vb_ref, kv_sems):
        my = lax.axis_index("i")
        left = lax.rem(my + N_DEV - 1, N_DEV)
        right = lax.rem(my + 1, N_DEV)

        kdma = pltpu.make_async_copy(
            kext_ref.at[0, :, pl.ds(my * HQ_SHARD, HQ_SHARD), :],
            kf_ref, kv_sems.at[0])
        vdma = pltpu.make_async_copy(
            vext_ref.at[0, :, pl.ds(my * HQ_SHARD, HQ_SHARD), :],
            vf_ref, kv_sems.at[1])
        kdma.start()
        vdma.start()

        barrier_sem = pltpu.get_barrier_semaphore()
        for nbr in (left, right):
            pl.semaphore_signal(barrier_sem, inc=1, device_id=(nbr,),
                                device_id_type=pl.DeviceIdType.MESH)
        pl.semaphore_wait(barrier_sem, 2)

        wqb_ref[...] = wq_ref[...].astype(jnp.bfloat16)
        wob_ref[...] = wo_ref[...].astype(jnp.bfloat16)
        kdma.wait()
        kb_ref[...] = (kf_ref[...] * SCALE).astype(jnp.bfloat16)
        vdma.wait()
        vb_ref[...] = vf_ref[...].astype(jnp.bfloat16)

        def cw_chunk(j):
            return lax.rem(my - j + N_DEV, N_DEV)

        def ccw_chunk(j):
            return N_DEV + lax.rem(my + j, N_DEV)

        iota_col = lax.broadcasted_iota(jnp.int32, (P2, 1), 0)
        HKV = SKV // 2
        cb_lo = lax.broadcasted_iota(jnp.int32, (P2, HKV), 1) // BLK
        cb_hi = HKV // BLK + lax.broadcasted_iota(
            jnp.int32, (CHUNK, HKV), 1) // BLK

        def compute_pair(j):
            ca = cw_chunk(j)
            cc = ccw_chunk(j)
            r0a = ca * CHUNK
            r0b = cc * CHUNK
            xp_ref[0:CHUNK, :] = x_ref[pl.ds(r0a, CHUNK), :].astype(
                jnp.bfloat16)
            xp_ref[CHUNK:P2, :] = x_ref[pl.ds(r0b, CHUNK), :].astype(
                jnp.bfloat16)
            qp = jnp.dot(xp_ref[...], wqb_ref[...],
                         preferred_element_type=jnp.float32)
            qb = qp.astype(jnp.bfloat16)
            roff = jnp.where(iota_col < CHUNK, r0a, r0b - CHUNK)
            rb = (iota_col + roff) // BLK
            bias1 = jnp.where(cb_lo <= rb, 0.0, -1e9)
            bias2 = jnp.where(cb_hi <= rb[CHUNK:P2, :], 0.0, -1e9)
            ctr = (((1,), (0,)), ((), ()))
            for h in range(HQ_SHARD):
                qh = qb[:, h * DH:(h + 1) * DH]
                s1 = lax.dot_general(
                    qh, kb_ref[0:HKV, h, :],
                    (((1,), (1,)), ((), ())),
                    preferred_element_type=jnp.float32)
                s2 = lax.dot_general(
                    qh[CHUNK:P2, :], kb_ref[HKV:SKV, h, :],
                    (((1,), (1,)), ((), ())),
                    preferred_element_type=jnp.float32)
                w1 = jnp.exp(s1 + bias1)
                w2 = jnp.exp(s2 + bias2)
                sum1 = jnp.sum(w1, axis=-1, keepdims=True)
                sum2 = jnp.sum(w2, axis=-1, keepdims=True)
                ctx1 = lax.dot_general(
                    w1.astype(jnp.bfloat16),
                    vb_ref[0:HKV, h, :], ctr,
                    preferred_element_type=jnp.float32)
                ctx2 = lax.dot_general(
                    w2.astype(jnp.bfloat16),
                    vb_ref[HKV:SKV, h, :], ctr,
                    preferred_element_type=jnp.float32)
                inv_a = 1.0 / sum1[0:CHUNK, :]
                inv_b = 1.0 / (sum1[CHUNK:P2, :] + sum2)
                ctx_ref[0:CHUNK, h * DH:(h + 1) * DH] = (
                    ctx1[0:CHUNK, :] * inv_a).astype(jnp.bfloat16)
                ctx_ref[CHUNK:P2, h * DH:(h + 1) * DH] = (
                    (ctx1[CHUNK:P2, :] + ctx2) * inv_b).astype(jnp.bfloat16)
            op = jnp.dot(ctx_ref[...], wob_ref[...],
                         preferred_element_type=jnp.float32)
            acc_ref[ca] = op[0:CHUNK, :].astype(jnp.bfloat16)
            acc_ref[cc] = op[CHUNK:P2, :].astype(jnp.bfloat16)

        def store_out(idx, val):
            out_ref[0, pl.ds(idx * CHUNK, CHUNK), :] = val

        if _SKIP_COMM:
            for j in range(N_DEV):
                compute_pair(j)
            for c in range(N_CHUNKS):
                store_out(c, acc_ref[c].astype(jnp.float32))
            return

        def make(src, slot, dst_dev):
            return pltpu.make_async_remote_copy(
                src_ref=src,
                dst_ref=comm_ref.at[slot],
                send_sem=send_sems.at[slot],
                recv_sem=recv_sems.at[slot],
                device_id=(dst_dev,),
                device_id_type=pl.DeviceIdType.MESH,
            )

        compute_pair(0)

        for hop in range(N_DEV - 1):
            cw = make(acc_ref.at[cw_chunk(hop)], hop, right)
            ccw = make(acc_ref.at[ccw_chunk(hop)], 3 + hop, left)
            cw.start()
            ccw.start()
            compute_pair(hop + 1)
            cw.wait_recv()
            ccw.wait_recv()
            cw_recv = cw_chunk(hop + 1)
            ccw_recv = ccw_chunk(hop + 1)
            acc_ref[cw_recv] = acc_ref[cw_recv] + comm_ref[hop]
            acc_ref[ccw_recv] = acc_ref[ccw_recv] + comm_ref[3 + hop]

        Rc = cw_chunk(N_DEV - 1)
        Sc = ccw_chunk(N_DEV - 1)
        step1 = [
            make(acc_ref.at[Rc], 6, right),
            make(acc_ref.at[Rc], 7, left),
            make(acc_ref.at[Sc], 8, right),
            make(acc_ref.at[Sc], 9, left),
        ]
        for r in step1:
            r.start()
        store_out(Rc, acc_ref[Rc].astype(jnp.float32))
        store_out(Sc, acc_ref[Sc].astype(jnp.float32))
        step1[0].wait_recv()
        step1[3].wait_recv()
        fwd_cw = make(comm_ref.at[6], 10, right)
        fwd_ccw = make(comm_ref.at[9], 11, left)
        fwd_cw.start()
        fwd_ccw.start()
        store_out(my, comm_ref[6].astype(jnp.float32))
        store_out(N_DEV + my, comm_ref[9].astype(jnp.float32))
        step1[1].wait_recv()
        step1[2].wait_recv()
        store_out(lax.rem(my + 2, N_DEV),
                  comm_ref[7].astype(jnp.float32))
        store_out(N_DEV + lax.rem(my + 2, N_DEV),
                  comm_ref[8].astype(jnp.float32))
        fwd_cw.wait_recv()
        fwd_ccw.wait_recv()
        store_out(lax.rem(my - 1 + N_DEV, N_DEV),
                  comm_ref[10].astype(jnp.float32))
        store_out(N_DEV + lax.rem(my + 1, N_DEV),
                  comm_ref[11].astype(jnp.float32))

        for slot in range(2 * N_HOPS):
            drain = make(comm_ref.at[slot], slot, right)
            drain.wait_send()

    out = pl.pallas_call(
        body,
        out_shape=jax.ShapeDtypeStruct((1, SQ, SQ), jnp.float32),
        in_specs=[
            pl.BlockSpec(memory_space=pltpu.VMEM),
            pl.BlockSpec(memory_space=pltpu.VMEM),
            pl.BlockSpec(memory_space=pltpu.MemorySpace.HBM),
            pl.BlockSpec(memory_space=pltpu.MemorySpace.HBM),
            pl.BlockSpec(memory_space=pltpu.VMEM),
        ],
        out_specs=pl.BlockSpec(memory_space=pltpu.VMEM),
        scratch_shapes=[
            pltpu.VMEM((N_CHUNKS, CHUNK, SQ), jnp.bfloat16),
            pltpu.VMEM((2 * N_HOPS, CHUNK, SQ), jnp.bfloat16),
            pltpu.SemaphoreType.DMA((2 * N_HOPS,)),
            pltpu.SemaphoreType.DMA((2 * N_HOPS,)),
            pltpu.VMEM((P2, HQ_SHARD * DH), jnp.bfloat16),
            pltpu.VMEM((P2, SQ), jnp.bfloat16),
            pltpu.VMEM((SQ, HQ_SHARD * DH), jnp.bfloat16),
            pltpu.VMEM((HQ_SHARD * DH, SQ), jnp.bfloat16),
            pltpu.VMEM((SKV, HQ_SHARD, DH), jnp.float32),
            pltpu.VMEM((SKV, HQ_SHARD, DH), jnp.float32),
            pltpu.VMEM((SKV, HQ_SHARD, DH), jnp.bfloat16),
            pltpu.VMEM((SKV, HQ_SHARD, DH), jnp.bfloat16),
            pltpu.SemaphoreType.DMA((2,)),
        ],
        compiler_params=pltpu.CompilerParams(collective_id=0),
    )(x2, Wq, K_ext, V_ext, Wo)
    return out


# device time: 46468 ns/iter; 1.5899x vs baseline; 1.5899x over previous
import os

import jax
import jax.numpy as jnp
from jax import lax
from jax.experimental import pallas as pl
from jax.experimental.pallas import tpu as pltpu

N_DEV = 4
SQ = 1024
SKV = 1024
HQ_SHARD = 8
DH = 128
BLK = 64
SCALE = 0.08838834764831843
N_CHUNKS = 2 * N_DEV
CHUNK = SQ // N_CHUNKS
N_HOPS = 2 * (N_DEV - 1)
P2 = 2 * CHUNK

_SKIP_COMM = bool(os.environ.get("SKIP_COMM"))


def kernel(x, Wq, K_ext, V_ext, Wo):
    my_i = lax.axis_index("i")
    x2 = x[0]
    K = lax.dynamic_slice_in_dim(K_ext[0], my_i * HQ_SHARD, HQ_SHARD, axis=1)
    V = lax.dynamic_slice_in_dim(V_ext[0], my_i * HQ_SHARD, HQ_SHARD, axis=1)
    Ks = (K * SCALE).astype(jnp.bfloat16).reshape(SKV, HQ_SHARD * DH)
    Vs = V.astype(jnp.bfloat16).reshape(SKV, HQ_SHARD * DH)

    def body(x_ref, wq_ref, k_ref, v_ref, wo_ref, out_ref,
             acc_ref, comm_ref, send_sems, recv_sems, ctx_ref, xp_ref,
             wqb_ref, wob_ref):
        my = lax.axis_index("i")
        left = lax.rem(my + N_DEV - 1, N_DEV)
        right = lax.rem(my + 1, N_DEV)

        barrier_sem = pltpu.get_barrier_semaphore()
        for nbr in (left, right):
            pl.semaphore_signal(barrier_sem, inc=1, device_id=(nbr,),
                                device_id_type=pl.DeviceIdType.MESH)
        pl.semaphore_wait(barrier_sem, 2)

        wqb_ref[...] = wq_ref[...].astype(jnp.bfloat16)
        wob_ref[...] = wo_ref[...].astype(jnp.bfloat16)

        def cw_chunk(j):
            return lax.rem(my - j + N_DEV, N_DEV)

        def ccw_chunk(j):
            return N_DEV + lax.rem(my + j, N_DEV)

        iota_col = lax.broadcasted_iota(jnp.int32, (P2, 1), 0)
        HKV = SKV // 2
        cb_lo = lax.broadcasted_iota(jnp.int32, (P2, HKV), 1) // BLK
        cb_hi = HKV // BLK + lax.broadcasted_iota(
            jnp.int32, (CHUNK, HKV), 1) // BLK

        def compute_pair(j):
            ca = cw_chunk(j)
            cc = ccw_chunk(j)
            r0a = ca * CHUNK
            r0b = cc * CHUNK
            xp_ref[0:CHUNK, :] = x_ref[pl.ds(r0a, CHUNK), :].astype(
                jnp.bfloat16)
            xp_ref[CHUNK:P2, :] = x_ref[pl.ds(r0b, CHUNK), :].astype(
                jnp.bfloat16)
            qp = jnp.dot(xp_ref[...], wqb_ref[...],
                         preferred_element_type=jnp.float32)
            qb = qp.astype(jnp.bfloat16)
            roff = jnp.where(iota_col < CHUNK, r0a, r0b - CHUNK)
            rb = (iota_col + roff) // BLK
            bias1 = jnp.where(cb_lo <= rb, 0.0, -1e9)
            bias2 = jnp.where(cb_hi <= rb[CHUNK:P2, :], 0.0, -1e9)
            ctr = (((1,), (0,)), ((), ()))
            for h in range(HQ_SHARD):
                qh = qb[:, h * DH:(h + 1) * DH]
                s1 = lax.dot_general(
                    qh, k_ref[0:HKV, h * DH:(h + 1) * DH],
                    (((1,), (1,)), ((), ())),
                    preferred_element_type=jnp.float32)
                s2 = lax.dot_general(
                    qh[CHUNK:P2, :], k_ref[HKV:SKV, h * DH:(h + 1) * DH],
                    (((1,), (1,)), ((), ())),
                    preferred_element_type=jnp.float32)
                w1 = jnp.exp(s1 + bias1)
                w2 = jnp.exp(s2 + bias2)
                sum1 = jnp.sum(w1, axis=-1, keepdims=True)
                sum2 = jnp.sum(w2, axis=-1, keepdims=True)
                ctx1 = lax.dot_general(
                    w1.astype(jnp.bfloat16),
                    v_ref[0:HKV, h * DH:(h + 1) * DH], ctr,
                    preferred_element_type=jnp.float32)
                ctx2 = lax.dot_general(
                    w2.astype(jnp.bfloat16),
                    v_ref[HKV:SKV, h * DH:(h + 1) * DH], ctr,
                    preferred_element_type=jnp.float32)
                inv_a = 1.0 / sum1[0:CHUNK, :]
                inv_b = 1.0 / (sum1[CHUNK:P2, :] + sum2)
                ctx_ref[0:CHUNK, h * DH:(h + 1) * DH] = (
                    ctx1[0:CHUNK, :] * inv_a).astype(jnp.bfloat16)
                ctx_ref[CHUNK:P2, h * DH:(h + 1) * DH] = (
                    (ctx1[CHUNK:P2, :] + ctx2) * inv_b).astype(jnp.bfloat16)
            op = jnp.dot(ctx_ref[...], wob_ref[...],
                         preferred_element_type=jnp.float32)
            acc_ref[ca] = op[0:CHUNK, :].astype(jnp.bfloat16)
            acc_ref[cc] = op[CHUNK:P2, :].astype(jnp.bfloat16)

        def store_out(idx, val):
            out_ref[pl.ds(idx * CHUNK, CHUNK), :] = val

        if _SKIP_COMM:
            for j in range(N_DEV):
                compute_pair(j)
            for c in range(N_CHUNKS):
                store_out(c, acc_ref[c].astype(jnp.float32))
            return

        def make(src, slot, dst_dev):
            return pltpu.make_async_remote_copy(
                src_ref=src,
                dst_ref=comm_ref.at[slot],
                send_sem=send_sems.at[slot],
                recv_sem=recv_sems.at[slot],
                device_id=(dst_dev,),
                device_id_type=pl.DeviceIdType.MESH,
            )

        compute_pair(0)

        for hop in range(N_DEV - 1):
            cw = make(acc_ref.at[cw_chunk(hop)], hop, right)
            ccw = make(acc_ref.at[ccw_chunk(hop)], 3 + hop, left)
            cw.start()
            ccw.start()
            compute_pair(hop + 1)
            cw.wait_recv()
            ccw.wait_recv()
            cw_recv = cw_chunk(hop + 1)
            ccw_recv = ccw_chunk(hop + 1)
            acc_ref[cw_recv] = acc_ref[cw_recv] + comm_ref[hop]
            acc_ref[ccw_recv] = acc_ref[ccw_recv] + comm_ref[3 + hop]

        Rc = cw_chunk(N_DEV - 1)
        Sc = ccw_chunk(N_DEV - 1)
        step1 = [
            make(acc_ref.at[Rc], 6, right),
            make(acc_ref.at[Rc], 7, left),
            make(acc_ref.at[Sc], 8, right),
            make(acc_ref.at[Sc], 9, left),
        ]
        for r in step1:
            r.start()
        store_out(Rc, acc_ref[Rc].astype(jnp.float32))
        store_out(Sc, acc_ref[Sc].astype(jnp.float32))
        step1[0].wait_recv()
        step1[3].wait_recv()
        fwd_cw = make(comm_ref.at[6], 10, right)
        fwd_ccw = make(comm_ref.at[9], 11, left)
        fwd_cw.start()
        fwd_ccw.start()
        store_out(my, comm_ref[6].astype(jnp.float32))
        store_out(N_DEV + my, comm_ref[9].astype(jnp.float32))
        step1[1].wait_recv()
        step1[2].wait_recv()
        store_out(lax.rem(my + 2, N_DEV),
                  comm_ref[7].astype(jnp.float32))
        store_out(N_DEV + lax.rem(my + 2, N_DEV),
                  comm_ref[8].astype(jnp.float32))
        fwd_cw.wait_recv()
        fwd_ccw.wait_recv()
        store_out(lax.rem(my - 1 + N_DEV, N_DEV),
                  comm_ref[10].astype(jnp.float32))
        store_out(N_DEV + lax.rem(my + 1, N_DEV),
                  comm_ref[11].astype(jnp.float32))

        for slot in range(2 * N_HOPS):
            drain = make(comm_ref.at[slot], slot, right)
            drain.wait_send()

    out = pl.pallas_call(
        body,
        out_shape=jax.ShapeDtypeStruct((SQ, SQ), jnp.float32),
        in_specs=[pl.BlockSpec(memory_space=pltpu.VMEM)] * 5,
        out_specs=pl.BlockSpec(memory_space=pltpu.VMEM),
        scratch_shapes=[
            pltpu.VMEM((N_CHUNKS, CHUNK, SQ), jnp.bfloat16),
            pltpu.VMEM((2 * N_HOPS, CHUNK, SQ), jnp.bfloat16),
            pltpu.SemaphoreType.DMA((2 * N_HOPS,)),
            pltpu.SemaphoreType.DMA((2 * N_HOPS,)),
            pltpu.VMEM((P2, HQ_SHARD * DH), jnp.bfloat16),
            pltpu.VMEM((P2, SQ), jnp.bfloat16),
            pltpu.VMEM((SQ, HQ_SHARD * DH), jnp.bfloat16),
            pltpu.VMEM((HQ_SHARD * DH, SQ), jnp.bfloat16),
        ],
        compiler_params=pltpu.CompilerParams(collective_id=0),
    )(x2, Wq, Ks, Vs, Wo)
    return out.reshape(1, SQ, SQ)
